# baseline (speedup 1.0000x reference)
import functools
import numpy as np
import jax
import jax.numpy as jnp
from jax import lax
from jax.experimental import pallas as pl
from jax.experimental.pallas import tpu as pltpu
from jax.experimental.pallas import tpu_sc as plsc

B, N, K, D = 16, 512, 32, 128
C = 64
NCHUNK = (B * N) // C
NW = 32
CHUNKS_PER_W = NCHUNK // NW
PAIRS = C * K
LANES = 16
GROUPS = PAIRS // LANES


def _sc_scatter_body(cols_hbm, vals_hbm, m_hbm,
                     idx_v, val_v, acc_v, sems):
    wid = lax.axis_index("s") * 2 + lax.axis_index("c")
    zeros = jnp.zeros((LANES,), jnp.float32)

    def dst(chunk):
        return m_hbm.at[chunk // (N // C),
                        pl.ds((chunk % (N // C)) * C, C), :]

    for cc in range(CHUNKS_PER_W):
        buf = cc % 2
        chunk = wid * CHUNKS_PER_W + cc
        base = chunk * PAIRS

        if cc >= 2:
            pltpu.make_async_copy(
                acc_v.at[buf], dst(chunk - 2), sems.at[buf]).wait()
            def unzero(g, carry):
                off = g * LANES
                cols = idx_v[buf, pl.ds(off, LANES)]
                rows = jnp.full((LANES,), g // (K // LANES), jnp.int32)
                plsc.store_scatter(acc_v.at[buf], [rows, cols], zeros)
                return carry
            lax.fori_loop(0, GROUPS, unzero, 0)
        else:
            def zero_blk(r, carry):
                for j in range(N // LANES):
                    acc_v[buf, r, pl.ds(j * LANES, LANES)] = zeros
                return carry
            lax.fori_loop(0, C, zero_blk, 0)

        pltpu.sync_copy(cols_hbm.at[pl.ds(base, PAIRS)], idx_v.at[buf])
        pltpu.sync_copy(vals_hbm.at[pl.ds(base, PAIRS)], val_v.at[buf])

        def group(g, carry):
            off = g * LANES
            cols = idx_v[buf, pl.ds(off, LANES)]
            vals = val_v[buf, pl.ds(off, LANES)]
            rows = jnp.full((LANES,), g // (K // LANES), jnp.int32)
            plsc.addupdate_scatter(acc_v.at[buf], [rows, cols], vals)
            return carry

        lax.fori_loop(0, GROUPS, group, 0)
        pltpu.async_copy(acc_v.at[buf], dst(chunk), sems.at[buf])

    for cc in range(CHUNKS_PER_W - 2, CHUNKS_PER_W):
        buf = cc % 2
        chunk = wid * CHUNKS_PER_W + cc
        pltpu.make_async_copy(
            acc_v.at[buf], dst(chunk), sems.at[buf]).wait()


def _build_m_sc(cols_flat, vals_flat):
    mesh = plsc.VectorSubcoreMesh(core_axis_name="c", subcore_axis_name="s",
                                  num_cores=2, num_subcores=16)
    k = pl.kernel(
        _sc_scatter_body,
        out_type=jax.ShapeDtypeStruct((B, N, N), jnp.float32),
        mesh=mesh,
        scratch_types=[
            pltpu.VMEM((2, PAIRS), jnp.int32),
            pltpu.VMEM((2, PAIRS), jnp.float32),
            pltpu.VMEM((2, C, N), jnp.float32),
            pltpu.SemaphoreType.DMA((2,)),
        ],
        compiler_params=pltpu.CompilerParams(
            needs_layout_passes=False, use_tc_tiling_on_sc=False),
    )
    return k(cols_flat, vals_flat)


def _mm_body(m_ref, h_ref, o_ref):
    m = m_ref[0]
    h1 = jnp.dot(m, h_ref[0], preferred_element_type=jnp.float32)
    o_ref[0] = jnp.dot(m, h1, preferred_element_type=jnp.float32)


def _two_layer_mm(m, h):
    return pl.pallas_call(
        _mm_body,
        grid=(B,),
        in_specs=[
            pl.BlockSpec((1, N, N), lambda b: (b, 0, 0)),
            pl.BlockSpec((1, N, D), lambda b: (b, 0, 0)),
        ],
        out_specs=pl.BlockSpec((1, N, D), lambda b: (b, 0, 0)),
        out_shape=jax.ShapeDtypeStruct((B, N, D), jnp.float32),
    )(m, h)


@jax.jit
def kernel(node_hidden, in_node_index, in_node_mask):
    cols_flat = in_node_index.reshape(-1)
    vals_flat = in_node_mask.reshape(-1)

    m = _build_m_sc(cols_flat, vals_flat)
    return _two_layer_mm(m, node_hidden)


# speedup vs baseline: 1.2887x; 1.2887x over previous
import functools
import numpy as np
import jax
import jax.numpy as jnp
from jax import lax
from jax.experimental import pallas as pl
from jax.experimental.pallas import tpu as pltpu
from jax.experimental.pallas import tpu_sc as plsc

B, N, K, D = 16, 512, 32, 128
C = 64
NCHUNK = (B * N) // C
NW = 32
CHUNKS_PER_W = NCHUNK // NW
PAIRS = C * K
LANES = 16
GROUPS = PAIRS // LANES


def _sc_scatter_body(cols_hbm, vals_hbm, m_hbm,
                     idx_v, val_v, acc_v, sems):
    wid = lax.axis_index("s") * 2 + lax.axis_index("c")
    zeros = jnp.zeros((LANES,), jnp.float32)

    def dst(chunk):
        return m_hbm.at[chunk // (N // C),
                        pl.ds((chunk % (N // C)) * (C // 8), C // 8)]

    for cc in range(CHUNKS_PER_W):
        buf = cc % 2
        chunk = wid * CHUNKS_PER_W + cc
        base = chunk * PAIRS

        if cc >= 2:
            pltpu.make_async_copy(
                acc_v.at[buf], dst(chunk - 2), sems.at[buf]).wait()
            def unzero(g, carry):
                off = g * LANES
                cols = idx_v[buf, pl.ds(off, LANES)]
                row = g // (K // LANES)
                rb = jnp.full((LANES,), row // 8, jnp.int32)
                r = jnp.full((LANES,), row % 8, jnp.int32)
                plsc.store_scatter(
                    acc_v.at[buf],
                    [rb, lax.shift_right_logical(cols, 7), r,
                     lax.bitwise_and(cols, 127)], zeros)
                return carry
            lax.fori_loop(0, GROUPS, unzero, 0)
        else:
            def zero_blk(i, carry):
                rb = i // 8
                r = i % 8
                for cb in range(N // 128):
                    for j in range(128 // LANES):
                        acc_v[buf, rb, cb, r, pl.ds(j * LANES, LANES)] = zeros
                return carry
            lax.fori_loop(0, C, zero_blk, 0)

        pltpu.sync_copy(cols_hbm.at[pl.ds(base, PAIRS)], idx_v.at[buf])
        pltpu.sync_copy(vals_hbm.at[pl.ds(base, PAIRS)], val_v.at[buf])

        def group(g, carry):
            off = g * LANES
            cols = idx_v[buf, pl.ds(off, LANES)]
            vals = val_v[buf, pl.ds(off, LANES)]
            row = g // (K // LANES)
            rb = jnp.full((LANES,), row // 8, jnp.int32)
            r = jnp.full((LANES,), row % 8, jnp.int32)
            plsc.addupdate_scatter(
                acc_v.at[buf],
                [rb, lax.shift_right_logical(cols, 7), r,
                 lax.bitwise_and(cols, 127)], vals)
            return carry

        lax.fori_loop(0, GROUPS, group, 0)
        pltpu.async_copy(acc_v.at[buf], dst(chunk), sems.at[buf])

    for cc in range(CHUNKS_PER_W - 2, CHUNKS_PER_W):
        buf = cc % 2
        chunk = wid * CHUNKS_PER_W + cc
        pltpu.make_async_copy(
            acc_v.at[buf], dst(chunk), sems.at[buf]).wait()


def _build_m_sc(cols_flat, vals_flat):
    mesh = plsc.VectorSubcoreMesh(core_axis_name="c", subcore_axis_name="s",
                                  num_cores=2, num_subcores=16)
    k = pl.kernel(
        _sc_scatter_body,
        out_type=jax.ShapeDtypeStruct((B, N // 8, N // 128, 8, 128),
                                      jnp.float32),
        mesh=mesh,
        scratch_types=[
            pltpu.VMEM((2, PAIRS), jnp.int32),
            pltpu.VMEM((2, PAIRS), jnp.float32),
            pltpu.VMEM((2, C // 8, N // 128, 8, 128), jnp.float32),
            pltpu.SemaphoreType.DMA((2,)),
        ],
        compiler_params=pltpu.CompilerParams(
            needs_layout_passes=False, use_tc_tiling_on_sc=False),
    )
    return k(cols_flat, vals_flat)


def _mm_body(m_ref, h_ref, o_ref):
    m = m_ref[0]
    h1 = jnp.dot(m, h_ref[0], preferred_element_type=jnp.float32)
    o_ref[0] = jnp.dot(m, h1, preferred_element_type=jnp.float32)


def _two_layer_mm(m, h):
    return pl.pallas_call(
        _mm_body,
        grid=(B,),
        in_specs=[
            pl.BlockSpec((1, N, N), lambda b: (b, 0, 0)),
            pl.BlockSpec((1, N, D), lambda b: (b, 0, 0)),
        ],
        out_specs=pl.BlockSpec((1, N, D), lambda b: (b, 0, 0)),
        out_shape=jax.ShapeDtypeStruct((B, N, D), jnp.float32),
    )(m, h)


@jax.jit
def kernel(node_hidden, in_node_index, in_node_mask):
    cols_flat = in_node_index.reshape(-1)
    vals_flat = in_node_mask.reshape(-1)

    m5 = _build_m_sc(cols_flat, vals_flat)
    m = m5.transpose(0, 1, 3, 2, 4).reshape(B, N, N)
    return _two_layer_mm(m, node_hidden)


# speedup vs baseline: 1.4354x; 1.1138x over previous
import functools
import numpy as np
import jax
import jax.numpy as jnp
from jax import lax
from jax.experimental import pallas as pl
from jax.experimental.pallas import tpu as pltpu
from jax.experimental.pallas import tpu_sc as plsc

B, N, K, D = 16, 512, 32, 128
C = 64
NCHUNK = (B * N) // C
NW = 32
CHUNKS_PER_W = NCHUNK // NW
PAIRS = C * K
LANES = 16
GROUPS = PAIRS // LANES


def _sc_scatter_body(cols_hbm, vals_hbm, m_hbm,
                     idx_v, val_v, acc_v, sems, sem_in):
    wid = lax.axis_index("s") * 2 + lax.axis_index("c")
    zeros = jnp.zeros((LANES,), jnp.float32)

    def dst(chunk):
        return m_hbm.at[chunk // (N // C),
                        pl.ds((chunk % (N // C)) * (C // 8), C // 8)]

    base = wid * CHUNKS_PER_W * PAIRS
    cin = pltpu.async_copy(
        cols_hbm.at[pl.ds(base, CHUNKS_PER_W * PAIRS)], idx_v, sem_in)
    vin = pltpu.async_copy(
        vals_hbm.at[pl.ds(base, CHUNKS_PER_W * PAIRS)], val_v, sem_in)

    def zero_blk(i, carry):
        rb = i // 8
        r = i % 8
        for buf in range(2):
            for cb in range(N // 128):
                for j in range(128 // LANES):
                    acc_v[buf, rb, cb, r, pl.ds(j * LANES, LANES)] = zeros
        return carry
    lax.fori_loop(0, C, zero_blk, 0)
    cin.wait()
    vin.wait()

    def scatter_idx(g, cols):
        row = g // (K // LANES)
        rb = jnp.full((LANES,), row // 8, jnp.int32)
        r = jnp.full((LANES,), row % 8, jnp.int32)
        return [rb, lax.shift_right_logical(cols, 7), r,
                lax.bitwise_and(cols, 127)]

    for cc in range(CHUNKS_PER_W):
        buf = cc % 2
        chunk = wid * CHUNKS_PER_W + cc
        off0 = cc * PAIRS

        if cc >= 2:
            pltpu.make_async_copy(
                acc_v.at[buf], dst(chunk - 2), sems.at[buf]).wait()
            prev0 = (cc - 2) * PAIRS
            def unzero(g, carry):
                cols = idx_v[pl.ds(prev0 + g * LANES, LANES)]
                plsc.store_scatter(acc_v.at[buf], scatter_idx(g, cols), zeros)
                return carry
            lax.fori_loop(0, GROUPS, unzero, 0)

        def group(g, carry):
            cols = idx_v[pl.ds(off0 + g * LANES, LANES)]
            vals = val_v[pl.ds(off0 + g * LANES, LANES)]
            plsc.addupdate_scatter(acc_v.at[buf], scatter_idx(g, cols), vals)
            return carry

        lax.fori_loop(0, GROUPS, group, 0)
        pltpu.async_copy(acc_v.at[buf], dst(chunk), sems.at[buf])

    for cc in range(CHUNKS_PER_W - 2, CHUNKS_PER_W):
        buf = cc % 2
        chunk = wid * CHUNKS_PER_W + cc
        pltpu.make_async_copy(
            acc_v.at[buf], dst(chunk), sems.at[buf]).wait()


def _build_m_sc(cols_flat, vals_flat):
    mesh = plsc.VectorSubcoreMesh(core_axis_name="c", subcore_axis_name="s",
                                  num_cores=2, num_subcores=16)
    k = pl.kernel(
        _sc_scatter_body,
        out_type=jax.ShapeDtypeStruct((B, N // 8, N // 128, 8, 128),
                                      jnp.float32),
        mesh=mesh,
        scratch_types=[
            pltpu.VMEM((CHUNKS_PER_W * PAIRS,), jnp.int32),
            pltpu.VMEM((CHUNKS_PER_W * PAIRS,), jnp.float32),
            pltpu.VMEM((2, C // 8, N // 128, 8, 128), jnp.float32),
            pltpu.SemaphoreType.DMA((2,)),
            pltpu.SemaphoreType.DMA,
        ],
        compiler_params=pltpu.CompilerParams(
            needs_layout_passes=False, use_tc_tiling_on_sc=False),
    )
    return k(cols_flat, vals_flat)


def _mm_body(m_ref, h_ref, o_ref):
    m = m_ref[0]
    h1 = jnp.dot(m, h_ref[0], preferred_element_type=jnp.float32)
    o_ref[0] = jnp.dot(m, h1, preferred_element_type=jnp.float32)


def _two_layer_mm(m, h):
    return pl.pallas_call(
        _mm_body,
        grid=(B,),
        in_specs=[
            pl.BlockSpec((1, N, N), lambda b: (b, 0, 0)),
            pl.BlockSpec((1, N, D), lambda b: (b, 0, 0)),
        ],
        out_specs=pl.BlockSpec((1, N, D), lambda b: (b, 0, 0)),
        out_shape=jax.ShapeDtypeStruct((B, N, D), jnp.float32),
    )(m, h)


@jax.jit
def kernel(node_hidden, in_node_index, in_node_mask):
    cols_flat = in_node_index.reshape(-1)
    vals_flat = in_node_mask.reshape(-1)

    m5 = _build_m_sc(cols_flat, vals_flat)
    m = m5.transpose(0, 1, 3, 2, 4).reshape(B, N, N)
    return _two_layer_mm(m, node_hidden)
